# Initial kernel scaffold; baseline (speedup 1.0000x reference)
import jax
import jax.numpy as jnp
from jax.experimental import pallas as pl


def kernel(x, embed_weight):
    raise NotImplementedError("write your pallas kernel here")


# speedup vs baseline: 1.9624x; 1.9624x over previous
import jax
import jax.numpy as jnp
from jax.experimental import pallas as pl


_TB = 512


def _add_kernel(x_ref, emb_ref, out_ref):
    out_ref[...] = x_ref[...] + emb_ref[...][None, :, :]


def kernel(x, embed_weight):
    B, T, D = x.shape
    tb = min(_TB, T)
    grid = (T // tb,)
    return pl.pallas_call(
        _add_kernel,
        grid=grid,
        in_specs=[
            pl.BlockSpec((B, tb, D), lambda i: (0, i, 0)),
            pl.BlockSpec((tb, D), lambda i: (i, 0)),
        ],
        out_specs=pl.BlockSpec((B, tb, D), lambda i: (0, i, 0)),
        out_shape=jax.ShapeDtypeStruct((B, T, D), x.dtype),
    )(x, embed_weight)
